# baseline (speedup 1.0000x reference)
import functools

import jax
import jax.numpy as jnp
from jax import lax
from jax.experimental import pallas as pl
from jax.experimental.pallas import tpu as pltpu
from jax.experimental.pallas import tpu_sc as plsc

SEQ = 200
NSEQ = 4096
DIM = 64
NC = 2
NS = 16
NW = NC * NS
BBLK = NSEQ // 128
UNITS = SEQ * BBLK
UPW = UNITS // NW
NBUF = 4


def _body(idxt_hbm, tok_hbm, pos_hbm, out_hbm,
          pos_v, rot_tbl, i0, i1, i2, i3, r0, r1, r2, r3, o0, o1, o2, o3,
          isem, gsem, osem):
    idxb = (i0, i1, i2, i3)
    rows = (r0, r1, r2, r3)
    outb = (o0, o1, o2, o3)
    wid = lax.axis_index("s") * NC + lax.axis_index("c")
    ubase = wid * UPW
    pltpu.sync_copy(pos_hbm, pos_v)

    iota = lax.iota(jnp.int32, 16)
    bvecs = [bg * 16 + iota for bg in range(8)]
    zvec = iota * 0
    for k in range(16):
        rot_tbl[k, pl.ds(0, 16)] = jnp.where(iota + k > 15, iota + k - 16,
                                             iota + k)

    def su(u):
        g = ubase + u
        return g // BBLK, g % BBLK

    def idx_cp(u, slot):
        s, bb = su(u)
        return pltpu.make_async_copy(
            idxt_hbm.at[s, pl.ds(bb * 128, 128)], idxb[slot], isem[slot])

    def gather_cp(u, slot):
        return pltpu.make_async_copy(
            tok_hbm.at[idxb[slot]], rows[slot], gsem[slot])

    def out_cps(u, slot):
        s, bb = su(u)
        return [pltpu.make_async_copy(
            outb[slot], out_hbm.at[s, :, bb], osem[slot])]

    def transpose_add(u, slot):
        s, _ = su(u)
        pbase = zvec + s * DIM

        @plsc.parallel_loop(0, DIM, 16)
        def dloop(d0):
            @plsc.parallel_loop(0, 16, 1, unroll=4)
            def kloop(k):
                dvec = rot_tbl[k, pl.ds(0, 16)] + d0
                dkv = lax.shift_right_logical(dvec, 3)
                div = lax.bitwise_and(dvec, 7)
                pv = plsc.load_gather(pos_v, [pbase + dvec])
                for bg in range(8):
                    v = plsc.load_gather(rows[slot], [bvecs[bg], dvec])
                    plsc.store_scatter(outb[slot], [dkv, div, bvecs[bg]], v + pv)

    def step(u, slot):
        gather_cp(u, slot).wait()
        transpose_add(u, slot)
        for cp in out_cps(u, slot):
            cp.start()

        @pl.when(u + NBUF < UPW)
        def _():
            idx_cp(u + NBUF, slot).start()

        sn = (slot + 2) % NBUF

        @pl.when(u >= 2)
        def _():
            for cp in out_cps(u - 2, sn):
                cp.wait()

        @pl.when(u + 2 < UPW)
        def _():
            idx_cp(u + 2, sn).wait()
            gather_cp(u + 2, sn).start()

    for u in range(NBUF):
        idx_cp(u, u).start()
    for u in range(2):
        idx_cp(u, u).wait()
        gather_cp(u, u).start()

    def group(k, carry):
        for j in range(NBUF):
            step(k * NBUF + j, j)
        return carry

    lax.fori_loop(0, UPW // NBUF, group, 0)

    for u in range(UPW - 2, UPW):
        for cp in out_cps(u, u % NBUF):
            cp.wait()


def kernel(inputs, token_table, pos_table):
    idx_t = inputs.astype(jnp.int32).T
    pos_flat = pos_table.reshape(-1)
    mesh = plsc.VectorSubcoreMesh(core_axis_name="c", subcore_axis_name="s")
    run = functools.partial(
        pl.kernel,
        mesh=mesh,
        compiler_params=pltpu.CompilerParams(
            use_tc_tiling_on_sc=False, needs_layout_passes=False),
        out_type=jax.ShapeDtypeStruct((SEQ, 8, BBLK, 8, 128), jnp.float32),
        scratch_types=[
            pltpu.VMEM((SEQ * DIM,), jnp.float32),
            pltpu.VMEM((16, 16), jnp.int32),
            pltpu.VMEM((128,), jnp.int32),
            pltpu.VMEM((128,), jnp.int32),
            pltpu.VMEM((128,), jnp.int32),
            pltpu.VMEM((128,), jnp.int32),
            pltpu.VMEM((128, DIM), jnp.float32),
            pltpu.VMEM((128, DIM), jnp.float32),
            pltpu.VMEM((128, DIM), jnp.float32),
            pltpu.VMEM((128, DIM), jnp.float32),
            pltpu.VMEM((8, 8, 128), jnp.float32),
            pltpu.VMEM((8, 8, 128), jnp.float32),
            pltpu.VMEM((8, 8, 128), jnp.float32),
            pltpu.VMEM((8, 8, 128), jnp.float32),
            [pltpu.SemaphoreType.DMA] * NBUF,
            [pltpu.SemaphoreType.DMA] * NBUF,
            [pltpu.SemaphoreType.DMA] * NBUF,
        ],
    )(_body)
    out5 = run(idx_t, token_table, pos_flat)
    return jnp.transpose(out5, (2, 4, 0, 1, 3)).reshape(NSEQ, SEQ, DIM)


# speedup vs baseline: 1.0526x; 1.0526x over previous
import functools

import jax
import jax.numpy as jnp
from jax import lax
from jax.experimental import pallas as pl
from jax.experimental.pallas import tpu as pltpu
from jax.experimental.pallas import tpu_sc as plsc

SEQ = 200
NSEQ = 4096
DIM = 64
NC = 2
NS = 16
NW = NC * NS
BBLK = NSEQ // 128
UNITS = SEQ * BBLK
UPW = UNITS // NW
NBUF = 4


def _body(idxt_hbm, tok_hbm, pos_hbm, out_hbm,
          pos_v, rot_tbl, i0, i1, i2, i3, r0, r1, r2, r3, o0, o1, o2, o3,
          isem, gsem, osem):
    idxb = (i0, i1, i2, i3)
    rows = (r0, r1, r2, r3)
    outb = (o0, o1, o2, o3)
    wid = lax.axis_index("s") * NC + lax.axis_index("c")
    ubase = wid * UPW
    pltpu.sync_copy(pos_hbm, pos_v)

    iota = lax.iota(jnp.int32, 16)
    bvecs = [bg * 16 + iota for bg in range(8)]
    zvec = iota * 0
    for k in range(16):
        rot_tbl[k, pl.ds(0, 16)] = jnp.where(iota + k > 15, iota + k - 16,
                                             iota + k)

    def su(u):
        g = ubase + u
        return g // BBLK, g % BBLK

    def idx_cp(u, slot):
        s, bb = su(u)
        return pltpu.make_async_copy(
            idxt_hbm.at[s, pl.ds(bb * 128, 128)], idxb[slot], isem[slot])

    def gather_cp(u, slot):
        return pltpu.make_async_copy(
            tok_hbm.at[idxb[slot]], rows[slot], gsem[slot])

    def out_cps(u, slot):
        s, bb = su(u)
        return [pltpu.make_async_copy(
            outb[slot].at[pl.ds(dk * 8, 8)], out_hbm.at[s, dk, bb], osem[slot])
            for dk in range(8)]

    def transpose_add(u, slot):
        s, _ = su(u)
        pbase = zvec + s * DIM

        @plsc.parallel_loop(0, DIM, 16)
        def dloop(d0):
            @plsc.parallel_loop(0, 16, 1, unroll=2)
            def kloop(k):
                dvec = rot_tbl[k, pl.ds(0, 16)] + d0
                pv = plsc.load_gather(pos_v, [pbase + dvec])
                for bg in range(8):
                    v = plsc.load_gather(rows[slot], [bvecs[bg], dvec])
                    plsc.store_scatter(outb[slot], [dvec, bvecs[bg]], v + pv)

    def step(u, slot):
        gather_cp(u, slot).wait()
        transpose_add(u, slot)
        for cp in out_cps(u, slot):
            cp.start()

        @pl.when(u + NBUF < UPW)
        def _():
            idx_cp(u + NBUF, slot).start()

        sn = (slot + 2) % NBUF

        @pl.when(u >= 2)
        def _():
            for cp in out_cps(u - 2, sn):
                cp.wait()

        @pl.when(u + 2 < UPW)
        def _():
            idx_cp(u + 2, sn).wait()
            gather_cp(u + 2, sn).start()

    for u in range(NBUF):
        idx_cp(u, u).start()
    for u in range(2):
        idx_cp(u, u).wait()
        gather_cp(u, u).start()

    def group(k, carry):
        for j in range(NBUF):
            step(k * NBUF + j, j)
        return carry

    lax.fori_loop(0, UPW // NBUF, group, 0)

    for u in range(UPW - 2, UPW):
        for cp in out_cps(u, u % NBUF):
            cp.wait()


def kernel(inputs, token_table, pos_table):
    idx_t = inputs.astype(jnp.int32).T
    pos_flat = pos_table.reshape(-1)
    mesh = plsc.VectorSubcoreMesh(core_axis_name="c", subcore_axis_name="s")
    run = functools.partial(
        pl.kernel,
        mesh=mesh,
        compiler_params=pltpu.CompilerParams(
            use_tc_tiling_on_sc=False, needs_layout_passes=False),
        out_type=jax.ShapeDtypeStruct((SEQ, 8, BBLK, 8, 128), jnp.float32),
        scratch_types=[
            pltpu.VMEM((SEQ * DIM,), jnp.float32),
            pltpu.VMEM((16, 16), jnp.int32),
            pltpu.VMEM((128,), jnp.int32),
            pltpu.VMEM((128,), jnp.int32),
            pltpu.VMEM((128,), jnp.int32),
            pltpu.VMEM((128,), jnp.int32),
            pltpu.VMEM((128, DIM), jnp.float32),
            pltpu.VMEM((128, DIM), jnp.float32),
            pltpu.VMEM((128, DIM), jnp.float32),
            pltpu.VMEM((128, DIM), jnp.float32),
            pltpu.VMEM((DIM, 128), jnp.float32),
            pltpu.VMEM((DIM, 128), jnp.float32),
            pltpu.VMEM((DIM, 128), jnp.float32),
            pltpu.VMEM((DIM, 128), jnp.float32),
            [pltpu.SemaphoreType.DMA] * NBUF,
            [pltpu.SemaphoreType.DMA] * NBUF,
            [pltpu.SemaphoreType.DMA] * NBUF,
        ],
    )(_body)
    out5 = run(idx_t, token_table, pos_flat)
    return jnp.transpose(out5, (2, 4, 0, 1, 3)).reshape(NSEQ, SEQ, DIM)
